# baseline (speedup 1.0000x reference)
import functools

import jax
import jax.numpy as jnp
from jax import lax
from jax.experimental import pallas as pl
from jax.experimental.pallas import tpu as pltpu
from jax.experimental.pallas import tpu_sc as plsc

_NUM_CLASSES = 10000
_D = 128
_N = 320000
_NC, _NS, _L = 2, 16, 16
_NW = _NC * _NS
_ROWS_PER_W = _N // _NW
_C = 80
_NCHUNK = _ROWS_PER_W // _C
_NBUF = 5
_MAIN_T = _NCHUNK // _NBUF - 1
_JREGS = _D // _L


@functools.partial(
    pl.kernel,
    out_type=(
        jax.ShapeDtypeStruct((_NW, _L), jnp.float32),
        jax.ShapeDtypeStruct((_N, _D), jnp.float32),
    ),
    mesh=plsc.VectorSubcoreMesh(
        core_axis_name="c", subcore_axis_name="s",
        num_cores=_NC, num_subcores=_NS),
    compiler_params=pltpu.CompilerParams(use_tc_tiling_on_sc=False),
    scratch_types=[
        pltpu.VMEM((_ROWS_PER_W,), jnp.int32),
        pltpu.VMEM((_NBUF, _C, _D), jnp.float32),
        pltpu.VMEM((_NBUF, _C, _D // 2), jnp.int32),
        pltpu.VMEM((_L,), jnp.float32),
    ] + [pltpu.SemaphoreType.DMA] * (3 * _NBUF),
)
def _sc_sqdist(x_hbm, lbl_hbm, tbl_hbm, out_hbm, outx_hbm,
               idx_all, x_v, m_v, acc_v, *sems):
    sem_x = sems[:_NBUF]
    sem_m = sems[_NBUF:2 * _NBUF]
    sem_w = sems[2 * _NBUF:3 * _NBUF]
    wid = lax.axis_index("s") * _NC + lax.axis_index("c")
    base = wid * _ROWS_PER_W

    pltpu.sync_copy(lbl_hbm.at[pl.ds(base, _ROWS_PER_W)], idx_all)

    def start_x(ci, b):
        row0 = base + ci * _C
        pltpu.async_copy(x_hbm.at[pl.ds(row0, _C)], x_v.at[b], sem_x[b])

    def start_m(ci, b):
        pltpu.async_copy(tbl_hbm.at[idx_all.at[pl.ds(ci * _C, _C)]],
                         m_v.at[b], sem_m[b])

    def start(ci, b):
        start_x(ci, b)
        start_m(ci, b)

    def wait(ci, b):
        row0 = base + ci * _C
        pltpu.make_async_copy(x_hbm.at[pl.ds(row0, _C)],
                              x_v.at[b], sem_x[b]).wait()
        pltpu.make_async_copy(tbl_hbm.at[pl.ds(0, _C)],
                              m_v.at[b], sem_m[b]).wait()

    def start_wb(ci, b):
        row0 = base + ci * _C
        pltpu.async_copy(x_v.at[b], outx_hbm.at[pl.ds(row0, _C)], sem_w[b])

    def wait_wb(ci, b):
        row0 = base + ci * _C
        pltpu.make_async_copy(x_v.at[b],
                              outx_hbm.at[pl.ds(row0, _C)], sem_w[b]).wait()

    def compute(b, accs):
        xb = x_v.at[b]
        mb = m_v.at[b]

        def row_body(r, a):
            new = list(a)
            for g in range(_JREGS // 2):
                mi = mb[r, pl.ds(g * _L, _L)]
                ma = jax.lax.bitcast_convert_type(mi << 16, jnp.float32)
                mc = jax.lax.bitcast_convert_type(mi & jnp.int32(-65536), jnp.float32)
                dva = xb[r, pl.ds((2 * g) * _L, _L)] - ma
                new[2 * g] = new[2 * g] + dva * dva
                dvc = xb[r, pl.ds((2 * g + 1) * _L, _L)] - mc
                new[2 * g + 1] = new[2 * g + 1] + dvc * dvc
            return tuple(new)

        return lax.fori_loop(0, _C, row_body, accs)

    for b in range(_NBUF):
        start_x(b, b)

    for b in range(_NBUF):
        start_m(b, b)

    def ring_turn(t, accs):
        for b in range(_NBUF):
            ci = t * _NBUF + b
            wait(ci, b)
            start_wb(ci, b)
            accs = compute(b, accs)
            wait_wb(ci, b)
            start(ci + _NBUF, b)
        return accs

    zero = jnp.zeros((_L,), jnp.float32)
    accs = lax.fori_loop(0, _MAIN_T, ring_turn, (zero,) * _JREGS)

    for b in range(_NBUF):
        ci = _MAIN_T * _NBUF + b
        wait(ci, b)
        start_wb(ci, b)
        accs = compute(b, accs)
        wait_wb(ci, b)

    total = accs[0]
    for j in range(1, _JREGS):
        total = total + accs[j]
    acc_v[...] = total
    pltpu.sync_copy(acc_v, out_hbm.at[wid])


def kernel(inputs, labels, running_mean):
    tbl_sw = (running_mean
              .reshape(_NUM_CLASSES, _D // 32, 2, _L)
              .transpose(0, 1, 3, 2)
              .reshape(_NUM_CLASSES, _D // 2, 2)
              .astype(jnp.bfloat16))
    tbl_i32 = jax.lax.bitcast_convert_type(tbl_sw, jnp.int32)
    partials, out_x = _sc_sqdist(inputs, labels.astype(jnp.int32), tbl_i32)
    regularization = jnp.sqrt(jnp.sum(partials))
    return out_x, regularization


# speedup vs baseline: 1.0063x; 1.0063x over previous
import functools

import jax
import jax.numpy as jnp
from jax import lax
from jax.experimental import pallas as pl
from jax.experimental.pallas import tpu as pltpu
from jax.experimental.pallas import tpu_sc as plsc

_NUM_CLASSES = 10000
_D = 128
_N = 320000
_NC, _NS, _L = 2, 16, 16
_NW = _NC * _NS
_ROWS_PER_W = _N // _NW
_C = 40
_NCHUNK = _ROWS_PER_W // _C
_NBUF = 5
_MAIN_T = _NCHUNK // _NBUF - 1
_JREGS = _D // _L


@functools.partial(
    pl.kernel,
    out_type=(
        jax.ShapeDtypeStruct((_NW, _L), jnp.float32),
        jax.ShapeDtypeStruct((_N, _D), jnp.float32),
    ),
    mesh=plsc.VectorSubcoreMesh(
        core_axis_name="c", subcore_axis_name="s",
        num_cores=_NC, num_subcores=_NS),
    compiler_params=pltpu.CompilerParams(use_tc_tiling_on_sc=False),
    scratch_types=[
        pltpu.VMEM((_ROWS_PER_W,), jnp.int32),
        pltpu.VMEM((_NBUF, _C, _D), jnp.float32),
        pltpu.VMEM((_NBUF, _C, _D // 2), jnp.int32),
        pltpu.VMEM((_L,), jnp.float32),
    ] + [pltpu.SemaphoreType.DMA] * (3 * _NBUF),
)
def _sc_sqdist(x_hbm, lbl_hbm, tbl_hbm, out_hbm, outx_hbm,
               idx_all, x_v, m_v, acc_v, *sems):
    sem_x = sems[:_NBUF]
    sem_m = sems[_NBUF:2 * _NBUF]
    sem_w = sems[2 * _NBUF:3 * _NBUF]
    wid = lax.axis_index("s") * _NC + lax.axis_index("c")
    base = wid * _ROWS_PER_W

    pltpu.sync_copy(lbl_hbm.at[pl.ds(base, _ROWS_PER_W)], idx_all)

    def start_x(ci, b):
        row0 = base + ci * _C
        pltpu.async_copy(x_hbm.at[pl.ds(row0, _C)], x_v.at[b], sem_x[b])

    def start_m(ci, b):
        pltpu.async_copy(tbl_hbm.at[idx_all.at[pl.ds(ci * _C, _C)]],
                         m_v.at[b], sem_m[b])

    def start(ci, b):
        start_x(ci, b)
        start_m(ci, b)

    def wait(ci, b):
        row0 = base + ci * _C
        pltpu.make_async_copy(x_hbm.at[pl.ds(row0, _C)],
                              x_v.at[b], sem_x[b]).wait()
        pltpu.make_async_copy(tbl_hbm.at[pl.ds(0, _C)],
                              m_v.at[b], sem_m[b]).wait()

    def start_wb(ci, b):
        row0 = base + ci * _C
        pltpu.async_copy(x_v.at[b], outx_hbm.at[pl.ds(row0, _C)], sem_w[b])

    def wait_wb(ci, b):
        row0 = base + ci * _C
        pltpu.make_async_copy(x_v.at[b],
                              outx_hbm.at[pl.ds(row0, _C)], sem_w[b]).wait()

    def compute(b, accs):
        xb = x_v.at[b]
        mb = m_v.at[b]

        def row_body(r, a):
            new = list(a)
            for g in range(_JREGS // 2):
                mi = mb[r, pl.ds(g * _L, _L)]
                ma = jax.lax.bitcast_convert_type(mi << 16, jnp.float32)
                mc = jax.lax.bitcast_convert_type(mi & jnp.int32(-65536), jnp.float32)
                dva = xb[r, pl.ds((2 * g) * _L, _L)] - ma
                new[2 * g] = new[2 * g] + dva * dva
                dvc = xb[r, pl.ds((2 * g + 1) * _L, _L)] - mc
                new[2 * g + 1] = new[2 * g + 1] + dvc * dvc
            return tuple(new)

        return lax.fori_loop(0, _C, row_body, accs)

    for b in range(_NBUF):
        start_x(b, b)

    for b in range(_NBUF):
        start_m(b, b)

    def ring_turn(t, accs):
        for b in range(_NBUF):
            ci = t * _NBUF + b
            wait(ci, b)
            start_wb(ci, b)
            accs = compute(b, accs)
            wait_wb(ci, b)
            start(ci + _NBUF, b)
        return accs

    zero = jnp.zeros((_L,), jnp.float32)
    accs = lax.fori_loop(0, _MAIN_T, ring_turn, (zero,) * _JREGS)

    for b in range(_NBUF):
        ci = _MAIN_T * _NBUF + b
        wait(ci, b)
        start_wb(ci, b)
        accs = compute(b, accs)
        wait_wb(ci, b)

    total = accs[0]
    for j in range(1, _JREGS):
        total = total + accs[j]
    acc_v[...] = total
    pltpu.sync_copy(acc_v, out_hbm.at[wid])


def kernel(inputs, labels, running_mean):
    tbl_sw = (running_mean
              .reshape(_NUM_CLASSES, _D // 32, 2, _L)
              .transpose(0, 1, 3, 2)
              .reshape(_NUM_CLASSES, _D // 2, 2)
              .astype(jnp.bfloat16))
    tbl_i32 = jax.lax.bitcast_convert_type(tbl_sw, jnp.int32)
    partials, out_x = _sc_sqdist(inputs, labels.astype(jnp.int32), tbl_i32)
    regularization = jnp.sqrt(jnp.sum(partials))
    return out_x, regularization


# speedup vs baseline: 1.0122x; 1.0058x over previous
import functools

import jax
import jax.numpy as jnp
from jax import lax
from jax.experimental import pallas as pl
from jax.experimental.pallas import tpu as pltpu
from jax.experimental.pallas import tpu_sc as plsc

_NUM_CLASSES = 10000
_D = 128
_N = 320000
_NC, _NS, _L = 2, 16, 16
_NW = _NC * _NS
_ROWS_PER_W = _N // _NW
_C = 40
_NCHUNK = _ROWS_PER_W // _C
_NBUF = 5
_MAIN_T = _NCHUNK // _NBUF - 1
_JREGS = _D // _L


@functools.partial(
    pl.kernel,
    out_type=(
        jax.ShapeDtypeStruct((_NW, _L), jnp.float32),
        jax.ShapeDtypeStruct((_N, _D), jnp.float32),
    ),
    mesh=plsc.VectorSubcoreMesh(
        core_axis_name="c", subcore_axis_name="s",
        num_cores=_NC, num_subcores=_NS),
    compiler_params=pltpu.CompilerParams(use_tc_tiling_on_sc=False),
    scratch_types=[
        pltpu.VMEM((_ROWS_PER_W,), jnp.int32),
        pltpu.VMEM((_NBUF, _C, _D), jnp.float32),
        pltpu.VMEM((_NBUF, _C, _D // 2), jnp.int32),
        pltpu.VMEM((_L,), jnp.float32),
    ] + [pltpu.SemaphoreType.DMA] * (3 * _NBUF),
)
def _sc_sqdist(x_hbm, lbl_hbm, tbl_hbm, out_hbm, outx_hbm,
               idx_all, x_v, m_v, acc_v, *sems):
    sem_x = sems[:_NBUF]
    sem_m = sems[_NBUF:2 * _NBUF]
    sem_w = sems[2 * _NBUF:3 * _NBUF]
    wid = lax.axis_index("s") * _NC + lax.axis_index("c")
    base = wid * _ROWS_PER_W

    pltpu.sync_copy(lbl_hbm.at[pl.ds(base, _ROWS_PER_W)], idx_all)

    def start_x(ci, b):
        row0 = base + ci * _C
        pltpu.async_copy(x_hbm.at[pl.ds(row0, _C)], x_v.at[b], sem_x[b])

    def start_m(ci, b):
        pltpu.async_copy(tbl_hbm.at[idx_all.at[pl.ds(ci * _C, _C)]],
                         m_v.at[b], sem_m[b])

    def start(ci, b):
        start_x(ci, b)
        start_m(ci, b)

    def wait(ci, b):
        row0 = base + ci * _C
        pltpu.make_async_copy(x_hbm.at[pl.ds(row0, _C)],
                              x_v.at[b], sem_x[b]).wait()
        pltpu.make_async_copy(tbl_hbm.at[pl.ds(0, _C)],
                              m_v.at[b], sem_m[b]).wait()

    def start_wb(ci, b):
        row0 = base + ci * _C
        pltpu.async_copy(x_v.at[b], outx_hbm.at[pl.ds(row0, _C)], sem_w[b])

    def wait_wb(ci, b):
        row0 = base + ci * _C
        pltpu.make_async_copy(x_v.at[b],
                              outx_hbm.at[pl.ds(row0, _C)], sem_w[b]).wait()

    def compute(b, accs):
        xb = x_v.at[b]
        mb = m_v.at[b]

        def row_body(r, a):
            new = list(a)
            for g in range(_JREGS // 2):
                mi = mb[r, pl.ds(g * _L, _L)]
                ma = jax.lax.bitcast_convert_type(mi << 16, jnp.float32)
                mc = jax.lax.bitcast_convert_type(mi & jnp.int32(-65536), jnp.float32)
                dva = xb[r, pl.ds((2 * g) * _L, _L)] - ma
                new[2 * g] = new[2 * g] + dva * dva
                dvc = xb[r, pl.ds((2 * g + 1) * _L, _L)] - mc
                new[2 * g + 1] = new[2 * g + 1] + dvc * dvc
            return tuple(new)

        return lax.fori_loop(0, _C, row_body, accs)

    for b in range(_NBUF):
        start_x(b, b)

    for b in range(_NBUF):
        start_m(b, b)

    def ring_turn(t, accs):
        for b in range(_NBUF):
            ci = t * _NBUF + b
            wait(ci, b)
            start_wb(ci, b)
            accs = compute(b, accs)
            wait_wb(ci, b)
            start(ci + _NBUF, b)
        return accs

    zero = jnp.zeros((_L,), jnp.float32)
    accs = lax.fori_loop(0, _MAIN_T, ring_turn, (zero,) * _JREGS)

    for b in range(_NBUF):
        ci = _MAIN_T * _NBUF + b
        wait(ci, b)
        start_wb(ci, b)
        accs = compute(b, accs)
        wait_wb(ci, b)

    total = accs[0]
    for j in range(1, _JREGS):
        total = total + accs[j]
    acc_v[...] = total
    pltpu.sync_copy(acc_v, out_hbm.at[wid])


def kernel(inputs, labels, running_mean):
    t4 = running_mean.reshape(_NUM_CLASSES, _D // 32, 2, _L)
    a16 = jax.lax.bitcast_convert_type(
        t4[:, :, 0, :].astype(jnp.bfloat16), jnp.uint16).astype(jnp.uint32)
    b16 = jax.lax.bitcast_convert_type(
        t4[:, :, 1, :].astype(jnp.bfloat16), jnp.uint16).astype(jnp.uint32)
    tbl_i32 = jax.lax.bitcast_convert_type(
        (b16 << 16) | a16, jnp.int32).reshape(_NUM_CLASSES, _D // 2)
    partials, out_x = _sc_sqdist(inputs, labels.astype(jnp.int32), tbl_i32)
    regularization = jnp.sqrt(jnp.sum(partials))
    return out_x, regularization


# speedup vs baseline: 1.1800x; 1.1659x over previous
import functools

import jax
import jax.numpy as jnp
from jax import lax
from jax.experimental import pallas as pl
from jax.experimental.pallas import tpu as pltpu
from jax.experimental.pallas import tpu_sc as plsc

_NUM_CLASSES = 10000
_D = 128
_N = 320000
_NC, _NS, _L = 2, 16, 16
_NW = _NC * _NS
_ROWS_PER_W = _N // _NW
_C = 40
_NCHUNK = _ROWS_PER_W // _C
_NBUF = 5
_MAIN_T = _NCHUNK // _NBUF - 1
_JREGS = _D // _L


@functools.partial(
    pl.kernel,
    out_type=(
        jax.ShapeDtypeStruct((_NW, _L), jnp.float32),
        jax.ShapeDtypeStruct((_N, _D), jnp.float32),
    ),
    mesh=plsc.VectorSubcoreMesh(
        core_axis_name="c", subcore_axis_name="s",
        num_cores=_NC, num_subcores=_NS),
    compiler_params=pltpu.CompilerParams(use_tc_tiling_on_sc=False),
    scratch_types=[
        pltpu.VMEM((_ROWS_PER_W,), jnp.int32),
        pltpu.VMEM((_NBUF, _C, _D), jnp.float32),
        pltpu.VMEM((_NBUF, _C, _D // 2), jnp.int32),
        pltpu.VMEM((_L,), jnp.float32),
        pltpu.VMEM_SHARED((_NUM_CLASSES, _D // 2), jnp.int32),
    ] + [pltpu.SemaphoreType.DMA] * (3 * _NBUF + 1),
)
def _sc_sqdist(x_hbm, lbl_hbm, tbl_hbm, out_hbm, outx_hbm,
               idx_all, x_v, m_v, acc_v, spm_tbl, *sems):
    sem_x = sems[:_NBUF]
    sem_m = sems[_NBUF:2 * _NBUF]
    sem_w = sems[2 * _NBUF:3 * _NBUF]
    sem_t = sems[3 * _NBUF]
    sid = lax.axis_index("s")
    wid = sid * _NC + lax.axis_index("c")
    base = wid * _ROWS_PER_W

    trow = sid * 624
    pltpu.async_copy(tbl_hbm.at[pl.ds(trow, 624)],
                     spm_tbl.at[pl.ds(trow, 624)], sem_t)

    pltpu.sync_copy(lbl_hbm.at[pl.ds(base, _ROWS_PER_W)], idx_all)

    def start_x(ci, b):
        row0 = base + ci * _C
        pltpu.async_copy(x_hbm.at[pl.ds(row0, _C)], x_v.at[b], sem_x[b])

    def start_m(ci, b):
        pltpu.async_copy(spm_tbl.at[idx_all.at[pl.ds(ci * _C, _C)]],
                         m_v.at[b], sem_m[b])

    def start(ci, b):
        start_x(ci, b)
        start_m(ci, b)

    def wait(ci, b):
        row0 = base + ci * _C
        pltpu.make_async_copy(x_hbm.at[pl.ds(row0, _C)],
                              x_v.at[b], sem_x[b]).wait()
        pltpu.make_async_copy(spm_tbl.at[pl.ds(0, _C)],
                              m_v.at[b], sem_m[b]).wait()

    def start_wb(ci, b):
        row0 = base + ci * _C
        pltpu.async_copy(x_v.at[b], outx_hbm.at[pl.ds(row0, _C)], sem_w[b])

    def wait_wb(ci, b):
        row0 = base + ci * _C
        pltpu.make_async_copy(x_v.at[b],
                              outx_hbm.at[pl.ds(row0, _C)], sem_w[b]).wait()

    def compute(b, accs):
        xb = x_v.at[b]
        mb = m_v.at[b]

        def row_body(r, a):
            new = list(a)
            for g in range(_JREGS // 2):
                mi = mb[r, pl.ds(g * _L, _L)]
                ma = jax.lax.bitcast_convert_type(mi << 16, jnp.float32)
                mc = jax.lax.bitcast_convert_type(mi & jnp.int32(-65536), jnp.float32)
                dva = xb[r, pl.ds((2 * g) * _L, _L)] - ma
                new[2 * g] = new[2 * g] + dva * dva
                dvc = xb[r, pl.ds((2 * g + 1) * _L, _L)] - mc
                new[2 * g + 1] = new[2 * g + 1] + dvc * dvc
            return tuple(new)

        return lax.fori_loop(0, _C, row_body, accs)

    for b in range(_NBUF):
        start_x(b, b)

    pltpu.make_async_copy(tbl_hbm.at[pl.ds(trow, 624)],
                          spm_tbl.at[pl.ds(trow, 624)], sem_t).wait()

    @pl.when(sid == 0)
    def _stage_tail():
        pltpu.sync_copy(tbl_hbm.at[pl.ds(16 * 624, 16)],
                        spm_tbl.at[pl.ds(16 * 624, 16)])

    plsc.subcore_barrier()

    for b in range(_NBUF):
        start_m(b, b)

    def ring_turn(t, accs):
        for b in range(_NBUF):
            ci = t * _NBUF + b
            wait(ci, b)
            start_wb(ci, b)
            accs = compute(b, accs)
            wait_wb(ci, b)
            start(ci + _NBUF, b)
        return accs

    zero = jnp.zeros((_L,), jnp.float32)
    accs = lax.fori_loop(0, _MAIN_T, ring_turn, (zero,) * _JREGS)

    for b in range(_NBUF):
        ci = _MAIN_T * _NBUF + b
        wait(ci, b)
        start_wb(ci, b)
        accs = compute(b, accs)
        wait_wb(ci, b)

    total = accs[0]
    for j in range(1, _JREGS):
        total = total + accs[j]
    acc_v[...] = total
    pltpu.sync_copy(acc_v, out_hbm.at[wid])


def kernel(inputs, labels, running_mean):
    t4 = running_mean.reshape(_NUM_CLASSES, _D // 32, 2, _L)
    a16 = jax.lax.bitcast_convert_type(
        t4[:, :, 0, :].astype(jnp.bfloat16), jnp.uint16).astype(jnp.uint32)
    b16 = jax.lax.bitcast_convert_type(
        t4[:, :, 1, :].astype(jnp.bfloat16), jnp.uint16).astype(jnp.uint32)
    tbl_i32 = jax.lax.bitcast_convert_type(
        (b16 << 16) | a16, jnp.int32).reshape(_NUM_CLASSES, _D // 2)
    partials, out_x = _sc_sqdist(inputs, labels.astype(jnp.int32), tbl_i32)
    regularization = jnp.sqrt(jnp.sum(partials))
    return out_x, regularization
